# baseline (speedup 1.0000x reference)
import functools

import jax
import jax.numpy as jnp
from jax.experimental import pallas as pl
from jax.experimental.pallas import tpu as pltpu

_TN = 256
_SW = 16
_G = _SW // 8


def _tree_min(vs):
    while len(vs) > 1:
        vs = [jnp.minimum(vs[i], vs[i + 1]) for i in range(0, len(vs) - 1, 2)] \
            + ([vs[-1]] if len(vs) % 2 else [])
    return vs[0]


def _tile_kernel(a_ref, br_ref, out1_ref, out2_ref, colacc_ref, *, n_i, m):
    i = pl.program_id(1)

    @pl.when(i == 0)
    def _init():
        colacc_ref[...] = jnp.full((8, m), jnp.inf, jnp.float32)

    for h in range(_TN // _SW):
        hs = h * _SW
        axb = jnp.broadcast_to(a_ref[0, 0, hs:hs + _SW, 0:1], (_SW, 128))
        ayb = jnp.broadcast_to(a_ref[0, 0, hs:hs + _SW, 1:2], (_SW, 128))
        azb = jnp.broadcast_to(a_ref[0, 0, hs:hs + _SW, 2:3], (_SW, 128))
        nab = jnp.broadcast_to(a_ref[0, 0, hs:hs + _SW, 3:4], (_SW, 128))
        ax = [axb[8 * g:8 * (g + 1), :] for g in range(_G)]
        ay = [ayb[8 * g:8 * (g + 1), :] for g in range(_G)]
        az = [azb[8 * g:8 * (g + 1), :] for g in range(_G)]
        na = [nab[8 * g:8 * (g + 1), :] for g in range(_G)]

        inf = jnp.full((8, 128), jnp.inf, jnp.float32)
        rowaccs = [inf] * _G
        for c in range(m // 128):
            cs = 128 * c
            bx = br_ref[0, 0, :, cs:cs + 128]
            by = br_ref[0, 1, :, cs:cs + 128]
            bz = br_ref[0, 2, :, cs:cs + 128]
            nb = br_ref[0, 3, :, cs:cs + 128]
            colf = []
            for g in range(_G):
                e = ax[g] * bx + nb
                e = ay[g] * by + e
                e = az[g] * bz + e
                f = e + na[g]
                colf.append(f)
                rowaccs[g] = jnp.minimum(rowaccs[g], f)
            cm = _tree_min(colf)
            colacc_ref[:, cs:cs + 128] = jnp.minimum(
                colacc_ref[:, cs:cs + 128], cm)

        rowacc = jnp.concatenate(rowaccs, axis=0)
        out1_ref[0, 0, 0, hs:hs + _SW] = jnp.min(rowacc, axis=1)

    @pl.when(i == n_i - 1)
    def _finish():
        out2_ref[0, 0, :] = jnp.min(colacc_ref[...], axis=0)


def _chamfer_fused(x1, x2):
    b, n, _ = x1.shape
    m = x2.shape[1]
    assert n % _TN == 0 and m % 128 == 0
    n_i = n // _TN

    na = jnp.sum(x1 * x1, axis=-1)
    nb = jnp.sum(x2 * x2, axis=-1)
    a = jnp.concatenate([-2.0 * x1, na[..., None]], axis=-1)
    a = a.reshape(b, n_i, _TN, 4)
    bt = jnp.concatenate([x2, nb[..., None]], axis=-1).transpose(0, 2, 1)
    br = jnp.broadcast_to(bt[:, :, None, :], (b, 4, 8, m))

    out1, out2 = pl.pallas_call(
        functools.partial(_tile_kernel, n_i=n_i, m=m),
        grid=(b, n_i),
        in_specs=[
            pl.BlockSpec((1, 1, _TN, 4), lambda bi, i: (bi, i, 0, 0)),
            pl.BlockSpec((1, 4, 8, m), lambda bi, i: (bi, 0, 0, 0)),
        ],
        out_specs=[
            pl.BlockSpec((1, 1, 1, _TN), lambda bi, i: (bi, i, 0, 0)),
            pl.BlockSpec((1, 1, m), lambda bi, i: (bi, 0, 0)),
        ],
        out_shape=[
            jax.ShapeDtypeStruct((b, n_i, 1, _TN), jnp.float32),
            jax.ShapeDtypeStruct((b, 1, m), jnp.float32),
        ],
        scratch_shapes=[pltpu.VMEM((8, m), jnp.float32)],
    )(a, br)
    return out1.reshape(b, n), out2.reshape(b, m)


def kernel(xyz1, xyz2):
    dist1, dist2 = _chamfer_fused(xyz1, xyz2)
    return (dist1, dist2)


# speedup vs baseline: 1.1131x; 1.1131x over previous
import functools

import jax
import jax.numpy as jnp
from jax.experimental import pallas as pl
from jax.experimental.pallas import tpu as pltpu

_TN = 256
_SW = 64
_G = _SW // 8


def _tree_min(vs):
    while len(vs) > 1:
        vs = [jnp.minimum(vs[i], vs[i + 1]) for i in range(0, len(vs) - 1, 2)] \
            + ([vs[-1]] if len(vs) % 2 else [])
    return vs[0]


def _tile_kernel(a_ref, br_ref, out1_ref, out2_ref, colacc_ref, *, n_i, m):
    i = pl.program_id(1)

    @pl.when(i == 0)
    def _init():
        colacc_ref[...] = jnp.full((8, m), jnp.inf, jnp.float32)

    for h in range(_TN // _SW):
        hs = h * _SW
        axb = jnp.broadcast_to(a_ref[0, 0, hs:hs + _SW, 0:1], (_SW, 128))
        ayb = jnp.broadcast_to(a_ref[0, 0, hs:hs + _SW, 1:2], (_SW, 128))
        azb = jnp.broadcast_to(a_ref[0, 0, hs:hs + _SW, 2:3], (_SW, 128))
        nab = jnp.broadcast_to(a_ref[0, 0, hs:hs + _SW, 3:4], (_SW, 128))
        ax = [axb[8 * g:8 * (g + 1), :] for g in range(_G)]
        ay = [ayb[8 * g:8 * (g + 1), :] for g in range(_G)]
        az = [azb[8 * g:8 * (g + 1), :] for g in range(_G)]
        na = [nab[8 * g:8 * (g + 1), :] for g in range(_G)]

        inf = jnp.full((8, 128), jnp.inf, jnp.float32)
        rowaccs = [inf] * _G
        for c in range(m // 128):
            cs = 128 * c
            bx = br_ref[0, 0, :, cs:cs + 128]
            by = br_ref[0, 1, :, cs:cs + 128]
            bz = br_ref[0, 2, :, cs:cs + 128]
            nb = br_ref[0, 3, :, cs:cs + 128]
            colf = []
            for g in range(_G):
                e = ax[g] * bx + nb
                e = ay[g] * by + e
                e = az[g] * bz + e
                f = e + na[g]
                colf.append(f)
                rowaccs[g] = jnp.minimum(rowaccs[g], f)
            cm = _tree_min(colf)
            colacc_ref[:, cs:cs + 128] = jnp.minimum(
                colacc_ref[:, cs:cs + 128], cm)

        rowacc = jnp.concatenate(rowaccs, axis=0)
        out1_ref[0, 0, 0, hs:hs + _SW] = jnp.min(rowacc, axis=1)

    @pl.when(i == n_i - 1)
    def _finish():
        out2_ref[0, 0, :] = jnp.min(colacc_ref[...], axis=0)


def _chamfer_fused(x1, x2):
    b, n, _ = x1.shape
    m = x2.shape[1]
    assert n % _TN == 0 and m % 128 == 0
    n_i = n // _TN

    na = jnp.sum(x1 * x1, axis=-1)
    nb = jnp.sum(x2 * x2, axis=-1)
    a = jnp.concatenate([-2.0 * x1, na[..., None]], axis=-1)
    a = a.reshape(b, n_i, _TN, 4)
    bt = jnp.concatenate([x2, nb[..., None]], axis=-1).transpose(0, 2, 1)
    br = jnp.broadcast_to(bt[:, :, None, :], (b, 4, 8, m))

    out1, out2 = pl.pallas_call(
        functools.partial(_tile_kernel, n_i=n_i, m=m),
        grid=(b, n_i),
        in_specs=[
            pl.BlockSpec((1, 1, _TN, 4), lambda bi, i: (bi, i, 0, 0)),
            pl.BlockSpec((1, 4, 8, m), lambda bi, i: (bi, 0, 0, 0)),
        ],
        out_specs=[
            pl.BlockSpec((1, 1, 1, _TN), lambda bi, i: (bi, i, 0, 0)),
            pl.BlockSpec((1, 1, m), lambda bi, i: (bi, 0, 0)),
        ],
        out_shape=[
            jax.ShapeDtypeStruct((b, n_i, 1, _TN), jnp.float32),
            jax.ShapeDtypeStruct((b, 1, m), jnp.float32),
        ],
        scratch_shapes=[pltpu.VMEM((8, m), jnp.float32)],
    )(a, br)
    return out1.reshape(b, n), out2.reshape(b, m)


def kernel(xyz1, xyz2):
    dist1, dist2 = _chamfer_fused(xyz1, xyz2)
    return (dist1, dist2)


# speedup vs baseline: 1.1223x; 1.0082x over previous
import functools

import jax
import jax.numpy as jnp
from jax.experimental import pallas as pl
from jax.experimental.pallas import tpu as pltpu

_TN = 256
_SW = 128
_G = _SW // 8


def _tree_min(vs):
    while len(vs) > 1:
        vs = [jnp.minimum(vs[i], vs[i + 1]) for i in range(0, len(vs) - 1, 2)] \
            + ([vs[-1]] if len(vs) % 2 else [])
    return vs[0]


def _tile_kernel(a_ref, br_ref, out1_ref, out2_ref, colacc_ref, *, n_i, m):
    i = pl.program_id(1)

    @pl.when(i == 0)
    def _init():
        colacc_ref[...] = jnp.full((8, m), jnp.inf, jnp.float32)

    for h in range(_TN // _SW):
        hs = h * _SW
        axb = jnp.broadcast_to(a_ref[0, 0, hs:hs + _SW, 0:1], (_SW, 128))
        ayb = jnp.broadcast_to(a_ref[0, 0, hs:hs + _SW, 1:2], (_SW, 128))
        azb = jnp.broadcast_to(a_ref[0, 0, hs:hs + _SW, 2:3], (_SW, 128))
        nab = jnp.broadcast_to(a_ref[0, 0, hs:hs + _SW, 3:4], (_SW, 128))
        ax = [axb[8 * g:8 * (g + 1), :] for g in range(_G)]
        ay = [ayb[8 * g:8 * (g + 1), :] for g in range(_G)]
        az = [azb[8 * g:8 * (g + 1), :] for g in range(_G)]
        na = [nab[8 * g:8 * (g + 1), :] for g in range(_G)]

        inf = jnp.full((8, 128), jnp.inf, jnp.float32)
        rowaccs = [inf] * _G
        for c in range(m // 128):
            cs = 128 * c
            bx = br_ref[0, 0, :, cs:cs + 128]
            by = br_ref[0, 1, :, cs:cs + 128]
            bz = br_ref[0, 2, :, cs:cs + 128]
            nb = br_ref[0, 3, :, cs:cs + 128]
            colf = []
            for g in range(_G):
                e = ax[g] * bx + nb
                e = ay[g] * by + e
                e = az[g] * bz + e
                f = e + na[g]
                colf.append(f)
                rowaccs[g] = jnp.minimum(rowaccs[g], f)
            cm = _tree_min(colf)
            colacc_ref[:, cs:cs + 128] = jnp.minimum(
                colacc_ref[:, cs:cs + 128], cm)

        rowacc = jnp.concatenate(rowaccs, axis=0)
        out1_ref[0, 0, 0, hs:hs + _SW] = jnp.min(rowacc, axis=1)

    @pl.when(i == n_i - 1)
    def _finish():
        out2_ref[0, 0, :] = jnp.min(colacc_ref[...], axis=0)


def _chamfer_fused(x1, x2):
    b, n, _ = x1.shape
    m = x2.shape[1]
    assert n % _TN == 0 and m % 128 == 0
    n_i = n // _TN

    na = jnp.sum(x1 * x1, axis=-1)
    nb = jnp.sum(x2 * x2, axis=-1)
    a = jnp.concatenate([-2.0 * x1, na[..., None]], axis=-1)
    a = a.reshape(b, n_i, _TN, 4)
    bt = jnp.concatenate([x2, nb[..., None]], axis=-1).transpose(0, 2, 1)
    br = jnp.broadcast_to(bt[:, :, None, :], (b, 4, 8, m))

    out1, out2 = pl.pallas_call(
        functools.partial(_tile_kernel, n_i=n_i, m=m),
        grid=(b, n_i),
        in_specs=[
            pl.BlockSpec((1, 1, _TN, 4), lambda bi, i: (bi, i, 0, 0)),
            pl.BlockSpec((1, 4, 8, m), lambda bi, i: (bi, 0, 0, 0)),
        ],
        out_specs=[
            pl.BlockSpec((1, 1, 1, _TN), lambda bi, i: (bi, i, 0, 0)),
            pl.BlockSpec((1, 1, m), lambda bi, i: (bi, 0, 0)),
        ],
        out_shape=[
            jax.ShapeDtypeStruct((b, n_i, 1, _TN), jnp.float32),
            jax.ShapeDtypeStruct((b, 1, m), jnp.float32),
        ],
        scratch_shapes=[pltpu.VMEM((8, m), jnp.float32)],
    )(a, br)
    return out1.reshape(b, n), out2.reshape(b, m)


def kernel(xyz1, xyz2):
    dist1, dist2 = _chamfer_fused(xyz1, xyz2)
    return (dist1, dist2)
